# baseline (speedup 1.0000x reference)
import functools

import jax
import jax.numpy as jnp
from jax import lax
from jax.experimental import pallas as pl
from jax.experimental.pallas import tpu as pltpu
from jax.experimental.pallas import tpu_sc as plsc

B = 16384
F = 26
V = 100000
D = 32
NUM = 13
H = 512

CHB = 2048


@functools.lru_cache(maxsize=None)
def _make_gather(plane_lo: int, planes: int):
    info = plsc.get_sparse_core_info()
    nw = info.num_cores * info.num_subcores
    ppw = planes // nw
    assert planes % nw == 0 and B % CHB == 0 and CHB % 16 == 0

    mesh = plsc.VectorSubcoreMesh(core_axis_name="c", subcore_axis_name="s")

    @functools.partial(
        pl.kernel,
        mesh=mesh,
        compiler_params=pltpu.CompilerParams(needs_layout_passes=False),
        out_type=jax.ShapeDtypeStruct((planes, B), jnp.float32),
        scratch_types=[
            pltpu.VMEM((V,), jnp.float32),
            pltpu.VMEM((CHB,), jnp.int32),
            pltpu.VMEM((CHB,), jnp.int32),
            pltpu.VMEM((B,), jnp.float32),
            pltpu.SemaphoreType.DMA,
            pltpu.SemaphoreType.DMA,
            pltpu.SemaphoreType.DMA,
        ],
    )
    def gather_k(xcat_hbm, tab_hbm, out_hbm, plane_v, idx0, idx1, res_v,
                 is0, is1, ssem):
        wid = lax.axis_index("s") * info.num_cores + lax.axis_index("c")
        ibufs, isems = (idx0, idx1), (is0, is1)
        n_chunks = B // CHB

        def idx_start(f, cj, par):
            pltpu.async_copy(
                xcat_hbm.at[f, pl.ds(cj * CHB, CHB)], ibufs[par], isems[par]
            )

        def idx_wait(par):
            pltpu.make_async_copy(
                xcat_hbm.at[0, pl.ds(0, CHB)], ibufs[par], isems[par]
            ).wait()

        def per_plane(pi, carry):
            q = wid * ppw + pi
            p = plane_lo + q
            f = p // D
            d = p % D
            idx_start(f, 0, 0)
            idx_start(f, 1, 1)
            pltpu.sync_copy(tab_hbm.at[f, d], plane_v)

            @pl.when(pi > 0)
            def _():
                pltpu.make_async_copy(res_v, out_hbm.at[q], ssem).wait()

            def pair(t, cc):
                for par in range(2):
                    cj = 2 * t + par
                    idx_wait(par)

                    @plsc.parallel_loop(0, CHB // 16, unroll=8)
                    def vec16(s, _par=par, _cj=cj):
                        iv = ibufs[_par][pl.ds(s * 16, 16)]
                        res_v[pl.ds(_cj * CHB + s * 16, 16)] = plsc.load_gather(
                            plane_v, [iv]
                        )

                    @pl.when(t < (n_chunks // 2) - 1)
                    def _(par=par, cj=cj):
                        idx_start(f, cj + 2, par)

                    pltpu.async_copy(
                        res_v.at[pl.ds(cj * CHB, CHB)],
                        out_hbm.at[q, pl.ds(cj * CHB, CHB)],
                        ssem,
                    )
                return cc

            lax.fori_loop(0, n_chunks // 2, pair, 0)
            return carry

        lax.fori_loop(0, ppw, per_plane, 0)
        pltpu.make_async_copy(res_v, out_hbm.at[0], ssem).wait()

    return gather_k


HP = (F * D) // 2


def _partial(emb_a, w1a):
    bm = 2048

    def body(e_r, w_r, o_r):
        o_r[...] = lax.dot_general(
            e_r[...], w_r[...], (((0,), (0,)), ((), ())),
            preferred_element_type=jnp.float32,
        )

    return pl.pallas_call(
        body,
        grid=(B // bm,),
        in_specs=[
            pl.BlockSpec((HP, bm), lambda i: (0, i)),
            pl.BlockSpec((HP, H), lambda i: (0, 0)),
        ],
        out_specs=pl.BlockSpec((bm, H), lambda i: (i, 0)),
        out_shape=jax.ShapeDtypeStruct((B, H), jnp.float32),
    )(emb_a, w1a)


def _mlp(emb, p1, x_num, W1e, W1n, b1, W2, b2, W3, b3):
    bm = 2048
    grid = (B // bm,)
    fd = HP

    def body(emb_r, p1_r, xn_r, w1e_r, w1n_r, b1_r, w2_r, b2_r, w3_r, b3_r,
             out_r):
        x1 = lax.dot_general(
            emb_r[...],
            w1e_r[...],
            (((0,), (0,)), ((), ())),
            preferred_element_type=jnp.float32,
        )
        x1 = x1 + p1_r[...]
        x1 = x1 + jnp.dot(xn_r[...], w1n_r[...], preferred_element_type=jnp.float32)
        h1 = jnp.maximum(x1 + b1_r[...], 0.0)
        h2 = jnp.maximum(
            jnp.dot(h1, w2_r[...], preferred_element_type=jnp.float32) + b2_r[...], 0.0
        )
        out_r[...] = (
            jnp.dot(h2, w3_r[...], preferred_element_type=jnp.float32) + b3_r[...]
        )

    full = lambda shape: pl.BlockSpec(shape, lambda i: (0, 0))
    out = pl.pallas_call(
        body,
        grid=grid,
        in_specs=[
            pl.BlockSpec((fd, bm), lambda i: (0, i)),
            pl.BlockSpec((bm, H), lambda i: (i, 0)),
            pl.BlockSpec((bm, NUM), lambda i: (i, 0)),
            full((fd, H)),
            full((NUM, H)),
            full((1, H)),
            full((H, H // 2)),
            full((1, H // 2)),
            full((H // 2, 1)),
            full((1, 1)),
        ],
        out_specs=pl.BlockSpec((bm, 1), lambda i: (i, 0)),
        out_shape=jax.ShapeDtypeStruct((B, 1), jnp.float32),
    )(emb, p1, x_num, W1e, W1n, b1, W2, b2, W3, b3)
    return out[:, 0]


def kernel(x_cat, x_num, tables, W1, b1, W2, b2, W3, b3):
    tab_t = jnp.transpose(tables, (0, 2, 1))
    xcat_t = x_cat.astype(jnp.int32).T
    emb_a = _make_gather(0, HP)(xcat_t, tab_t)
    emb_b = _make_gather(HP, HP)(xcat_t, tab_t)
    p1 = _partial(emb_a, W1[:HP])
    return _mlp(
        emb_b,
        p1,
        x_num,
        W1[HP : F * D],
        W1[F * D :],
        b1.reshape(1, H),
        W2,
        b2.reshape(1, H // 2),
        W3,
        b3.reshape(1, 1),
    )


# speedup vs baseline: 1.0769x; 1.0769x over previous
import functools

import jax
import jax.numpy as jnp
from jax import lax
from jax.experimental import pallas as pl
from jax.experimental.pallas import tpu as pltpu
from jax.experimental.pallas import tpu_sc as plsc

B = 16384
F = 26
V = 100000
D = 32
NUM = 13
H = 512

CHB = 2048


@functools.lru_cache(maxsize=None)
def _make_gather():
    info = plsc.get_sparse_core_info()
    nw = info.num_cores * info.num_subcores
    planes = F * D
    ppw = planes // nw
    assert planes % nw == 0 and B % CHB == 0 and CHB % 16 == 0

    mesh = plsc.VectorSubcoreMesh(core_axis_name="c", subcore_axis_name="s")

    @functools.partial(
        pl.kernel,
        mesh=mesh,
        compiler_params=pltpu.CompilerParams(needs_layout_passes=False),
        out_type=jax.ShapeDtypeStruct((planes, B), jnp.float32),
        scratch_types=[
            pltpu.VMEM((V,), jnp.float32),
            pltpu.VMEM((CHB,), jnp.int32),
            pltpu.VMEM((CHB,), jnp.int32),
            pltpu.VMEM((B,), jnp.float32),
            pltpu.SemaphoreType.DMA,
            pltpu.SemaphoreType.DMA,
            pltpu.SemaphoreType.DMA,
        ],
    )
    def gather_k(xcat_hbm, tab_hbm, out_hbm, plane_v, idx0, idx1, res_v,
                 is0, is1, ssem):
        wid = lax.axis_index("s") * info.num_cores + lax.axis_index("c")
        ibufs, isems = (idx0, idx1), (is0, is1)
        n_chunks = B // CHB

        def idx_start(f, cj, par):
            pltpu.async_copy(
                xcat_hbm.at[f, pl.ds(cj * CHB, CHB)], ibufs[par], isems[par]
            )

        def idx_wait(par):
            pltpu.make_async_copy(
                xcat_hbm.at[0, pl.ds(0, CHB)], ibufs[par], isems[par]
            ).wait()

        def per_plane(pi, carry):
            p = wid * ppw + pi
            f = p // D
            d = p % D
            idx_start(f, 0, 0)
            idx_start(f, 1, 1)
            pltpu.sync_copy(tab_hbm.at[f, d], plane_v)

            @pl.when(pi > 0)
            def _():
                pltpu.make_async_copy(res_v, out_hbm.at[p], ssem).wait()

            def pair(t, cc):
                for par in range(2):
                    cj = 2 * t + par
                    idx_wait(par)

                    @plsc.parallel_loop(0, CHB // 16, unroll=8)
                    def vec16(s, _par=par, _cj=cj):
                        iv = ibufs[_par][pl.ds(s * 16, 16)]
                        res_v[pl.ds(_cj * CHB + s * 16, 16)] = plsc.load_gather(
                            plane_v, [iv]
                        )

                    @pl.when(t < (n_chunks // 2) - 1)
                    def _(par=par, cj=cj):
                        idx_start(f, cj + 2, par)

                    pltpu.async_copy(
                        res_v.at[pl.ds(cj * CHB, CHB)],
                        out_hbm.at[p, pl.ds(cj * CHB, CHB)],
                        ssem,
                    )
                return cc

            lax.fori_loop(0, n_chunks // 2, pair, 0)
            return carry

        lax.fori_loop(0, ppw, per_plane, 0)
        pltpu.make_async_copy(res_v, out_hbm.at[0], ssem).wait()

    return gather_k


def _mlp(emb, x_num, W1e, W1n, b1, W2, b2, W3, b3):
    bm = 2048
    grid = (B // bm,)
    fd = F * D

    def body(emb_r, xn_r, w1e_r, w1n_r, b1_r, w2_r, b2_r, w3_r, b3_r, out_r):
        x1 = lax.dot_general(
            emb_r[...],
            w1e_r[...],
            (((0,), (0,)), ((), ())),
            preferred_element_type=jnp.float32,
        )
        x1 = x1 + jnp.dot(xn_r[...], w1n_r[...], preferred_element_type=jnp.float32)
        h1 = jnp.maximum(x1 + b1_r[...], 0.0)
        h2 = jnp.maximum(
            jnp.dot(h1, w2_r[...], preferred_element_type=jnp.float32) + b2_r[...], 0.0
        )
        out_r[...] = (
            jnp.dot(h2, w3_r[...], preferred_element_type=jnp.float32) + b3_r[...]
        )

    full = lambda shape: pl.BlockSpec(shape, lambda i: (0, 0))
    out = pl.pallas_call(
        body,
        grid=grid,
        in_specs=[
            pl.BlockSpec((fd, bm), lambda i: (0, i)),
            pl.BlockSpec((bm, NUM), lambda i: (i, 0)),
            full((fd, H)),
            full((NUM, H)),
            full((1, H)),
            full((H, H // 2)),
            full((1, H // 2)),
            full((H // 2, 1)),
            full((1, 1)),
        ],
        out_specs=pl.BlockSpec((bm, 1), lambda i: (i, 0)),
        out_shape=jax.ShapeDtypeStruct((B, 1), jnp.float32),
    )(emb, x_num, W1e, W1n, b1, W2, b2, W3, b3)
    return out[:, 0]


def kernel(x_cat, x_num, tables, W1, b1, W2, b2, W3, b3):
    tab_t = jnp.transpose(tables, (0, 2, 1))
    xcat_t = x_cat.astype(jnp.int32).T
    emb_t = _make_gather()(xcat_t, tab_t)
    return _mlp(
        emb_t,
        x_num,
        W1[: F * D],
        W1[F * D :],
        b1.reshape(1, H),
        W2,
        b2.reshape(1, H // 2),
        W3,
        b3.reshape(1, 1),
    )


# speedup vs baseline: 1.1884x; 1.1035x over previous
import functools

import jax
import jax.numpy as jnp
from jax import lax
from jax.experimental import pallas as pl
from jax.experimental.pallas import tpu as pltpu
from jax.experimental.pallas import tpu_sc as plsc

B = 16384
F = 26
V = 100000
D = 32
NUM = 13
H = 512

CHB = 4096


@functools.lru_cache(maxsize=None)
def _make_gather():
    info = plsc.get_sparse_core_info()
    nw = info.num_cores * info.num_subcores
    planes = F * D
    ppw = planes // nw
    assert planes % nw == 0 and B % CHB == 0 and CHB % 16 == 0

    mesh = plsc.VectorSubcoreMesh(core_axis_name="c", subcore_axis_name="s")

    @functools.partial(
        pl.kernel,
        mesh=mesh,
        compiler_params=pltpu.CompilerParams(needs_layout_passes=False),
        out_type=jax.ShapeDtypeStruct((planes, B), jnp.float32),
        scratch_types=[
            pltpu.VMEM((V,), jnp.float32),
            pltpu.VMEM((CHB,), jnp.int32),
            pltpu.VMEM((CHB,), jnp.int32),
            pltpu.VMEM((B,), jnp.float32),
            pltpu.SemaphoreType.DMA,
            pltpu.SemaphoreType.DMA,
            pltpu.SemaphoreType.DMA,
        ],
    )
    def gather_k(xcat_hbm, tab_hbm, out_hbm, plane_v, idx0, idx1, res_v,
                 is0, is1, ssem):
        wid = lax.axis_index("s") * info.num_cores + lax.axis_index("c")
        ibufs, isems = (idx0, idx1), (is0, is1)
        n_chunks = B // CHB

        def idx_start(f, cj, par):
            pltpu.async_copy(
                xcat_hbm.at[f, pl.ds(cj * CHB, CHB)], ibufs[par], isems[par]
            )

        def idx_wait(par):
            pltpu.make_async_copy(
                xcat_hbm.at[0, pl.ds(0, CHB)], ibufs[par], isems[par]
            ).wait()

        def per_plane(pi, carry):
            p = wid * ppw + pi
            f = p // D
            d = p % D
            idx_start(f, 0, 0)
            idx_start(f, 1, 1)
            pltpu.sync_copy(tab_hbm.at[f, d], plane_v)

            @pl.when(pi > 0)
            def _():
                pltpu.make_async_copy(res_v, out_hbm.at[p], ssem).wait()

            def pair(t, cc):
                for par in range(2):
                    cj = 2 * t + par
                    idx_wait(par)

                    @plsc.parallel_loop(0, CHB // 16, unroll=8)
                    def vec16(s, _par=par, _cj=cj):
                        iv = ibufs[_par][pl.ds(s * 16, 16)]
                        res_v[pl.ds(_cj * CHB + s * 16, 16)] = plsc.load_gather(
                            plane_v, [iv]
                        )

                    @pl.when(t < (n_chunks // 2) - 1)
                    def _(par=par, cj=cj):
                        idx_start(f, cj + 2, par)

                    pltpu.async_copy(
                        res_v.at[pl.ds(cj * CHB, CHB)],
                        out_hbm.at[p, pl.ds(cj * CHB, CHB)],
                        ssem,
                    )
                return cc

            lax.fori_loop(0, n_chunks // 2, pair, 0)
            return carry

        lax.fori_loop(0, ppw, per_plane, 0)
        pltpu.make_async_copy(res_v, out_hbm.at[0], ssem).wait()

    return gather_k


def _mlp(emb, x_num, W1e, W1n, b1, W2, b2, W3, b3):
    bm = 2048
    grid = (B // bm,)
    fd = F * D

    def body(emb_r, xn_r, w1e_r, w1n_r, b1_r, w2_r, b2_r, w3_r, b3_r, out_r):
        x1 = lax.dot_general(
            emb_r[...],
            w1e_r[...],
            (((0,), (0,)), ((), ())),
            preferred_element_type=jnp.float32,
        )
        x1 = x1 + jnp.dot(xn_r[...], w1n_r[...], preferred_element_type=jnp.float32)
        h1 = jnp.maximum(x1 + b1_r[...], 0.0)
        h2 = jnp.maximum(
            jnp.dot(h1, w2_r[...], preferred_element_type=jnp.float32) + b2_r[...], 0.0
        )
        out_r[...] = (
            jnp.dot(h2, w3_r[...], preferred_element_type=jnp.float32) + b3_r[...]
        )

    full = lambda shape: pl.BlockSpec(shape, lambda i: (0, 0))
    out = pl.pallas_call(
        body,
        grid=grid,
        in_specs=[
            pl.BlockSpec((fd, bm), lambda i: (0, i)),
            pl.BlockSpec((bm, NUM), lambda i: (i, 0)),
            full((fd, H)),
            full((NUM, H)),
            full((1, H)),
            full((H, H // 2)),
            full((1, H // 2)),
            full((H // 2, 1)),
            full((1, 1)),
        ],
        out_specs=pl.BlockSpec((bm, 1), lambda i: (i, 0)),
        out_shape=jax.ShapeDtypeStruct((B, 1), jnp.float32),
    )(emb, x_num, W1e, W1n, b1, W2, b2, W3, b3)
    return out[:, 0]


def kernel(x_cat, x_num, tables, W1, b1, W2, b2, W3, b3):
    tab_t = jnp.transpose(tables, (0, 2, 1))
    xcat_t = x_cat.astype(jnp.int32).T
    emb_t = _make_gather()(xcat_t, tab_t)
    return _mlp(
        emb_t,
        x_num,
        W1[: F * D],
        W1[F * D :],
        b1.reshape(1, H),
        W2,
        b2.reshape(1, H // 2),
        W3,
        b3.reshape(1, 1),
    )


# speedup vs baseline: 1.1885x; 1.0002x over previous
import functools

import jax
import jax.numpy as jnp
from jax import lax
from jax.experimental import pallas as pl
from jax.experimental.pallas import tpu as pltpu
from jax.experimental.pallas import tpu_sc as plsc

B = 16384
F = 26
V = 100000
D = 32
NUM = 13
H = 512

CHB = 4096


@functools.lru_cache(maxsize=None)
def _make_gather():
    info = plsc.get_sparse_core_info()
    nw = info.num_cores * info.num_subcores
    planes = F * D
    ppw = planes // nw
    assert planes % nw == 0 and B % CHB == 0 and CHB % 16 == 0

    mesh = plsc.VectorSubcoreMesh(core_axis_name="c", subcore_axis_name="s")

    @functools.partial(
        pl.kernel,
        mesh=mesh,
        compiler_params=pltpu.CompilerParams(needs_layout_passes=False),
        out_type=jax.ShapeDtypeStruct((planes, B), jnp.float32),
        scratch_types=[
            pltpu.VMEM((V,), jnp.float32),
            pltpu.VMEM((CHB,), jnp.int32),
            pltpu.VMEM((CHB,), jnp.int32),
            pltpu.VMEM((B,), jnp.float32),
            pltpu.SemaphoreType.DMA,
            pltpu.SemaphoreType.DMA,
            pltpu.SemaphoreType.DMA,
        ],
    )
    def gather_k(xcat_hbm, tab_hbm, out_hbm, plane_v, idx0, idx1, res_v,
                 is0, is1, ssem):
        wid = lax.axis_index("s") * info.num_cores + lax.axis_index("c")
        ibufs, isems = (idx0, idx1), (is0, is1)
        n_chunks = B // CHB

        def idx_start(f, cj, par):
            pltpu.async_copy(
                xcat_hbm.at[f, pl.ds(cj * CHB, CHB)], ibufs[par], isems[par]
            )

        def idx_wait(par):
            pltpu.make_async_copy(
                xcat_hbm.at[0, pl.ds(0, CHB)], ibufs[par], isems[par]
            ).wait()

        def per_plane(pi, carry):
            p = wid * ppw + pi
            f = p // D
            d = p % D
            idx_start(f, 0, 0)
            idx_start(f, 1, 1)
            pltpu.sync_copy(tab_hbm.at[f, d], plane_v)

            @pl.when(pi > 0)
            def _():
                pltpu.make_async_copy(res_v, out_hbm.at[p], ssem).wait()

            def pair(t, cc):
                for par in range(2):
                    cj = 2 * t + par
                    idx_wait(par)

                    @plsc.parallel_loop(0, CHB // 16, unroll=8)
                    def vec16(s, _par=par, _cj=cj):
                        iv = ibufs[_par][pl.ds(s * 16, 16)]
                        res_v[pl.ds(_cj * CHB + s * 16, 16)] = plsc.load_gather(
                            plane_v, [iv]
                        )

                    @pl.when(t < (n_chunks // 2) - 1)
                    def _(par=par, cj=cj):
                        idx_start(f, cj + 2, par)

                    pltpu.async_copy(
                        res_v.at[pl.ds(cj * CHB, CHB)],
                        out_hbm.at[p, pl.ds(cj * CHB, CHB)],
                        ssem,
                    )
                return cc

            lax.fori_loop(0, n_chunks // 2, pair, 0)
            return carry

        lax.fori_loop(0, ppw, per_plane, 0)
        pltpu.make_async_copy(res_v, out_hbm.at[0], ssem).wait()

    return gather_k


def _mlp(emb, x_num, W1e, W1n, b1, W2, b2, W3, b3):
    bm = 4096
    grid = (B // bm,)
    fd = F * D

    def body(emb_r, xn_r, w1e_r, w1n_r, b1_r, w2_r, b2_r, w3_r, b3_r, out_r):
        x1 = lax.dot_general(
            emb_r[...],
            w1e_r[...],
            (((0,), (0,)), ((), ())),
            preferred_element_type=jnp.float32,
        )
        x1 = x1 + jnp.dot(xn_r[...], w1n_r[...], preferred_element_type=jnp.float32)
        h1 = jnp.maximum(x1 + b1_r[...], 0.0)
        h2 = jnp.maximum(
            jnp.dot(h1, w2_r[...], preferred_element_type=jnp.float32) + b2_r[...], 0.0
        )
        out_r[...] = (
            jnp.dot(h2, w3_r[...], preferred_element_type=jnp.float32) + b3_r[...]
        )

    full = lambda shape: pl.BlockSpec(shape, lambda i: (0, 0))
    out = pl.pallas_call(
        body,
        grid=grid,
        in_specs=[
            pl.BlockSpec((fd, bm), lambda i: (0, i)),
            pl.BlockSpec((bm, NUM), lambda i: (i, 0)),
            full((fd, H)),
            full((NUM, H)),
            full((1, H)),
            full((H, H // 2)),
            full((1, H // 2)),
            full((H // 2, 1)),
            full((1, 1)),
        ],
        out_specs=pl.BlockSpec((bm, 1), lambda i: (i, 0)),
        out_shape=jax.ShapeDtypeStruct((B, 1), jnp.float32),
    )(emb, x_num, W1e, W1n, b1, W2, b2, W3, b3)
    return out[:, 0]


def kernel(x_cat, x_num, tables, W1, b1, W2, b2, W3, b3):
    tab_t = jnp.transpose(tables, (0, 2, 1))
    xcat_t = x_cat.astype(jnp.int32).T
    emb_t = _make_gather()(xcat_t, tab_t)
    return _mlp(
        emb_t,
        x_num,
        W1[: F * D],
        W1[F * D :],
        b1.reshape(1, H),
        W2,
        b2.reshape(1, H // 2),
        W3,
        b3.reshape(1, 1),
    )
